# baseline (speedup 1.0000x reference)
import jax
import jax.numpy as jnp
from jax.experimental import pallas as pl
from jax.experimental.pallas import tpu as pltpu

N_CLS = 19
LB_POS = 1.0 - 0.1
LB_NEG = 0.1 / N_CLS
IGN = 255


def _body(lab_ref, out_ref):
    c = pl.program_id(1)
    lab = lab_ref[...]
    val = jnp.where(lab == c, LB_POS, LB_NEG)
    val = jnp.where(lab == IGN, 0.0, val)
    out_ref[...] = val[:, None, :, :]


def kernel(label):
    n, h, w = label.shape
    return pl.pallas_call(
        _body,
        grid=(n, N_CLS),
        in_specs=[pl.BlockSpec((1, h, w), lambda i, c: (i, 0, 0))],
        out_specs=pl.BlockSpec((1, 1, h, w), lambda i, c: (i, c, 0, 0)),
        out_shape=jax.ShapeDtypeStruct((n, N_CLS, h, w), jnp.float32),
        compiler_params=pltpu.CompilerParams(
            dimension_semantics=("arbitrary", "arbitrary"),
        ),
    )(label)


# speedup vs baseline: 1.7118x; 1.7118x over previous
import jax
import jax.numpy as jnp
from jax.experimental import pallas as pl
from jax.experimental.pallas import tpu as pltpu

N_CLS = 19
LB_POS = 1.0 - 0.1
LB_NEG = 0.1 / N_CLS
IGN = 255


def _body(lab_ref, out_ref):
    lab = lab_ref[...]
    neg = jnp.where(lab == IGN, 0.0, LB_NEG)
    for c in range(N_CLS):
        out_ref[:, c, :, :] = jnp.where(lab == c, LB_POS, neg)


def kernel(label):
    n, h, w = label.shape
    hb = 128
    return pl.pallas_call(
        _body,
        grid=(n, h // hb),
        in_specs=[pl.BlockSpec((1, hb, w), lambda i, j: (i, j, 0))],
        out_specs=pl.BlockSpec((1, N_CLS, hb, w), lambda i, j: (i, 0, j, 0)),
        out_shape=jax.ShapeDtypeStruct((n, N_CLS, h, w), jnp.float32),
        compiler_params=pltpu.CompilerParams(
            dimension_semantics=("arbitrary", "arbitrary"),
        ),
    )(label)
